# baseline (speedup 1.0000x reference)
import functools

import jax
import jax.numpy as jnp
from jax import lax
from jax.experimental import pallas as pl
from jax.experimental.pallas import tpu as pltpu
from jax.experimental.pallas import tpu_sc as plsc

V = 1000000
D = 32
J = 100000

SH = 1 << 17
NSTR = 8
V2 = NSTR * SH
TW = 4096
TGRID = SH // TW
NCB = -(-V // TW)
PKW = D // 2

NC = 2
NS = 16
L = 16
NW = NC * NS
PW = 3136
JPAD = NW * PW
C = 784
NCHUNK = PW // C

_mesh = plsc.VectorSubcoreMesh(core_axis_name="c", subcore_axis_name="s")


def _tc_body(*refs):
    ins, out = refs[:NSTR], refs[NSTR]
    z = jnp.concatenate([r[...] for r in ins], axis=0)
    z = pltpu.bitcast(z.astype(jnp.bfloat16), jnp.int32).T
    out[...] = z.reshape(TW * D * NSTR // 2)


def _stripe_map(b):
    def imap(i):
        return (0, jnp.minimum(b * TGRID + i, NCB - 1))
    return imap


_tc_transpose = pl.pallas_call(
    _tc_body,
    grid=(TGRID,),
    compiler_params=pltpu.CompilerParams(
        dimension_semantics=("arbitrary",),
    ),
    in_specs=[pl.BlockSpec((D, TW), _stripe_map(b)) for b in range(NSTR)],
    out_specs=pl.BlockSpec((TW * D * NSTR // 2,), lambda i: (i,)),
    out_shape=jax.ShapeDtypeStruct((V2 * PKW,), jnp.int32),
)


@functools.partial(
    pl.kernel,
    mesh=_mesh,
    compiler_params=pltpu.CompilerParams(use_tc_tiling_on_sc=False,
                                         needs_layout_passes=False),
    out_type=jax.ShapeDtypeStruct((JPAD, PKW), jnp.int32),
    scratch_types=[
        pltpu.VMEM((PW,), jnp.int32),
        pltpu.VMEM((2, C, PKW), jnp.int32),
        pltpu.SemaphoreType.DMA,
        pltpu.SemaphoreType.DMA,
        pltpu.SemaphoreType.DMA,
        pltpu.SemaphoreType.DMA,
    ],
)
def _sc_gather(ta, ia, out, idx_a, ra, sg0, sg1, so0, so1):
    wid = lax.axis_index("s") * NC + lax.axis_index("c")
    base = wid * PW

    pltpu.sync_copy(ia.at[pl.ds(base, PW)], idx_a)

    gsems = (sg0, sg1, so0, so1)

    def gcopy(k, slot):
        return pltpu.async_copy(ta.at[idx_a.at[pl.ds(k * C, C)]],
                                ra.at[slot], gsems[slot])

    gh = [gcopy(0, 0), gcopy(1, 1)]
    for k in range(NCHUNK):
        slot = k % 2
        gh[slot].wait()
        pltpu.sync_copy(ra.at[slot],
                        out.at[pl.ds(base + k * C, C), :])
        if k + 2 < NCHUNK:
            gh[slot] = gcopy(k + 2, slot)


@functools.partial(
    pl.kernel,
    mesh=_mesh,
    compiler_params=pltpu.CompilerParams(use_tc_tiling_on_sc=False,
                                         needs_layout_passes=False),
    out_type=jax.ShapeDtypeStruct((NW, L), jnp.float32),
    scratch_types=[
        pltpu.VMEM((PW,), jnp.int32),
        pltpu.VMEM((2, C, PKW), jnp.int32),
        pltpu.VMEM((2, C, PKW), jnp.int32),
        pltpu.VMEM((L,), jnp.float32),
        pltpu.SemaphoreType.DMA,
        pltpu.SemaphoreType.DMA,
        pltpu.SemaphoreType.DMA,
        pltpu.SemaphoreType.DMA,
    ],
)
def _sc_mse(ga, tb, ib, out, idx_b, ra, rb, outv, sa0, sa1, sb0, sb1):
    wid = lax.axis_index("s") * NC + lax.axis_index("c")
    base = wid * PW

    pltpu.sync_copy(ib.at[pl.ds(base, PW)], idx_b)

    sems_a = (sa0, sa1)
    sems_b = (sb0, sb1)

    def fire(k, slot):
        cpa = pltpu.async_copy(
            ga.at[pl.ds(base + k * C, C), :],
            ra.at[slot], sems_a[slot])
        cpb = pltpu.async_copy(tb.at[idx_b.at[pl.ds(k * C, C)]], rb.at[slot],
                               sems_b[slot])
        return cpa, cpb

    inflight = [fire(0, 0), fire(1, 1)]

    def chunk_sum(k, slot, acc):
        def body(r, accs):
            a0, a1 = accs
            g = base + k * C + r
            s = jnp.where(g < J, jnp.float32(1.0), jnp.float32(0.0))
            xa0, xa1 = plsc.unpack(
                plsc.bitcast(ra[slot, r, :], jnp.bfloat16),
                format=plsc.PackFormat.INTERLEAVED,
                preferred_element_type=jnp.float32)
            xb0, xb1 = plsc.unpack(
                plsc.bitcast(rb[slot, r, :], jnp.bfloat16),
                format=plsc.PackFormat.INTERLEAVED,
                preferred_element_type=jnp.float32)
            d0 = (xb0 - xa0) * s
            d1 = (xb1 - xa1) * s
            return a0 + d0 * d0, a1 + d1 * d1

        return lax.fori_loop(0, C, body, acc)

    acc = (jnp.zeros((L,), jnp.float32), jnp.zeros((L,), jnp.float32))
    for k in range(NCHUNK):
        slot = k % 2
        cpa, cpb = inflight[slot]
        cpa.wait()
        cpb.wait()
        acc = chunk_sum(k, slot, acc)
        if k + 2 < NCHUNK:
            inflight[slot] = fire(k + 2, slot)

    outv[...] = acc[0] + acc[1]
    pltpu.sync_copy(outv, out.at[wid])


def _permute_idx(idx):
    return ((idx & (SH - 1)) << 3) | (idx >> 17)


def kernel(wemb_i2t, wemb_nmt, idx_i2t, idx_nmt):
    pad = JPAD - J
    zpad = jnp.zeros((pad,), jnp.int32)
    ia = _permute_idx(jnp.concatenate([idx_i2t, zpad]))
    ib = _permute_idx(jnp.concatenate([idx_nmt, zpad]))
    flat_a = _tc_transpose(*([wemb_i2t.T] * NSTR))
    ga = _sc_gather(flat_a.reshape(V2, PKW), ia)
    flat_b = _tc_transpose(*([wemb_nmt.T] * NSTR))
    partials = _sc_mse(ga, flat_b.reshape(V2, PKW), ib)
    return jnp.sum(partials) / jnp.float32(J * D)


# speedup vs baseline: 1.0844x; 1.0844x over previous
import functools

import jax
import jax.numpy as jnp
from jax import lax
from jax.experimental import pallas as pl
from jax.experimental.pallas import tpu as pltpu
from jax.experimental.pallas import tpu_sc as plsc

V = 1000000
D = 32
J = 100000

SH = 1 << 17
NSTR = 8
V2 = NSTR * SH
TW = 4096
TGRID = SH // TW
NCB = -(-V // TW)
PKW = D // 2

NC = 2
NS = 16
L = 16
NW = NC * NS
PW = 3136
JPAD = NW * PW
C = 784
NCHUNK = PW // C

_mesh = plsc.VectorSubcoreMesh(core_axis_name="c", subcore_axis_name="s")


def _tc_body(*refs):
    ins, bins = refs[:NSTR], refs[NSTR:2 * NSTR]
    oa, ob = refs[2 * NSTR], refs[2 * NSTR + 1]
    za = jnp.concatenate([r[...] for r in ins], axis=0)
    za = pltpu.bitcast(za.astype(jnp.bfloat16), jnp.int32).T
    oa[...] = za.reshape(TW * D * NSTR // 2)
    zb = jnp.concatenate([r[...] for r in bins], axis=0)
    zb = pltpu.bitcast(zb.astype(jnp.bfloat16), jnp.int32).T
    ob[...] = zb.reshape(TW * D * NSTR // 2)


def _stripe_map(b):
    def imap(i):
        return (0, jnp.minimum(b * TGRID + i, NCB - 1))
    return imap


_tc_transpose = pl.pallas_call(
    _tc_body,
    grid=(TGRID,),
    compiler_params=pltpu.CompilerParams(
        dimension_semantics=("arbitrary",),
    ),
    in_specs=[pl.BlockSpec((D, TW), _stripe_map(b))
              for b in range(NSTR)] * 2,
    out_specs=[
        pl.BlockSpec((TW * D * NSTR // 2,), lambda i: (i,)),
        pl.BlockSpec((TW * D * NSTR // 2,), lambda i: (i,)),
    ],
    out_shape=[
        jax.ShapeDtypeStruct((V2 * PKW,), jnp.int32),
        jax.ShapeDtypeStruct((V2 * PKW,), jnp.int32),
    ],
)


@functools.partial(
    pl.kernel,
    mesh=_mesh,
    compiler_params=pltpu.CompilerParams(use_tc_tiling_on_sc=False,
                                         needs_layout_passes=False),
    out_type=jax.ShapeDtypeStruct((NW, L), jnp.float32),
    scratch_types=[
        pltpu.VMEM((PW,), jnp.int32),
        pltpu.VMEM((PW,), jnp.int32),
        pltpu.VMEM((2, C, PKW), jnp.int32),
        pltpu.VMEM((2, C, PKW), jnp.int32),
        pltpu.VMEM((L,), jnp.float32),
        pltpu.SemaphoreType.DMA,
        pltpu.SemaphoreType.DMA,
        pltpu.SemaphoreType.DMA,
        pltpu.SemaphoreType.DMA,
    ],
)
def _sc_mse(ta, tb, ia, ib, out, idx_a, idx_b, ra, rb, outv,
            sa0, sa1, sb0, sb1):
    wid = lax.axis_index("s") * NC + lax.axis_index("c")
    base = wid * PW

    pltpu.sync_copy(ia.at[pl.ds(base, PW)], idx_a)
    pltpu.sync_copy(ib.at[pl.ds(base, PW)], idx_b)

    sems_a = (sa0, sa1)
    sems_b = (sb0, sb1)

    def fire(k, slot):
        cpa = pltpu.async_copy(ta.at[idx_a.at[pl.ds(k * C, C)]], ra.at[slot],
                               sems_a[slot])
        cpb = pltpu.async_copy(tb.at[idx_b.at[pl.ds(k * C, C)]], rb.at[slot],
                               sems_b[slot])
        return cpa, cpb

    inflight = [fire(0, 0), fire(1, 1)]

    def chunk_sum(k, slot, acc):
        def body(r, accs):
            a0, a1 = accs
            g = base + k * C + r
            s = jnp.where(g < J, jnp.float32(1.0), jnp.float32(0.0))
            xa0, xa1 = plsc.unpack(
                plsc.bitcast(ra[slot, r, :], jnp.bfloat16),
                format=plsc.PackFormat.INTERLEAVED,
                preferred_element_type=jnp.float32)
            xb0, xb1 = plsc.unpack(
                plsc.bitcast(rb[slot, r, :], jnp.bfloat16),
                format=plsc.PackFormat.INTERLEAVED,
                preferred_element_type=jnp.float32)
            d0 = (xb0 - xa0) * s
            d1 = (xb1 - xa1) * s
            return a0 + d0 * d0, a1 + d1 * d1

        return lax.fori_loop(0, C, body, acc)

    acc = (jnp.zeros((L,), jnp.float32), jnp.zeros((L,), jnp.float32))
    for k in range(NCHUNK):
        slot = k % 2
        cpa, cpb = inflight[slot]
        cpa.wait()
        cpb.wait()
        acc = chunk_sum(k, slot, acc)
        if k + 2 < NCHUNK:
            inflight[slot] = fire(k + 2, slot)

    outv[...] = acc[0] + acc[1]
    pltpu.sync_copy(outv, out.at[wid])


def kernel(wemb_i2t, wemb_nmt, idx_i2t, idx_nmt):
    flat_a, flat_b = _tc_transpose(*([wemb_i2t.T] * NSTR + [wemb_nmt.T] * NSTR))
    ta = flat_a.reshape(V2, PKW)
    tb = flat_b.reshape(V2, PKW)
    pad = JPAD - J
    zpad = jnp.zeros((pad,), jnp.int32)
    ia = jnp.concatenate([idx_i2t, zpad])
    ib = jnp.concatenate([idx_nmt, zpad])
    ia = ((ia & (SH - 1)) << 3) | (ia >> 17)
    ib = ((ib & (SH - 1)) << 3) | (ib >> 17)
    partials = _sc_mse(ta, tb, ia, ib)
    return jnp.sum(partials) / jnp.float32(J * D)
